# baseline (speedup 1.0000x reference)
import functools

import jax
import jax.numpy as jnp
from jax import lax
from jax.experimental import pallas as pl
from jax.experimental.pallas import tpu as pltpu
from jax.experimental.pallas import tpu_sc as plsc

V = 1000000
D = 64
B = 16384
NN = 5
NW = 32
BPW = B // NW
NCH = BPW // 128


def _sc_kernel():
    mesh = plsc.VectorSubcoreMesh(core_axis_name="c", subcore_axis_name="s")

    @functools.partial(
        pl.kernel,
        mesh=mesh,
        compiler_params=pltpu.CompilerParams(needs_layout_passes=False),
        out_type=jax.ShapeDtypeStruct((NN, B // 128, 128), jnp.float32),
        scratch_types=[
            pltpu.VMEM((NCH, 128), jnp.int32),
            pltpu.VMEM((NCH, 128), jnp.int32),
            pltpu.VMEM((NCH, 128), jnp.int32),
            pltpu.VMEM((BPW, 128), jnp.float32),
            pltpu.VMEM((128, 128), jnp.float32),
            pltpu.VMEM((NCH, 128), jnp.float32),
            pltpu.SemaphoreType.DMA,
        ],
    )
    def k(tgt_hbm, ctx_hbm, wt_hbm, wc_hbm, out_hbm, idx_t, idx_c, idx_h,
          rows_t, rows_c, dots_v, sem):
        wid = lax.axis_index("s") * 2 + lax.axis_index("c")
        crow = wid * NCH
        lanes = lax.iota(jnp.int32, 16)

        def halve(src, dst):
            for r in range(NCH):
                for c in range(128 // 16):
                    dst[r, pl.ds(c * 16, 16)] = (
                        src[r, pl.ds(c * 16, 16)] >> 1)

        pltpu.sync_copy(tgt_hbm.at[pl.ds(crow, NCH)], idx_t)
        halve(idx_t, idx_h)
        for j in range(NCH):
            pltpu.async_copy(
                wt_hbm.at[idx_h.at[j]],
                rows_t.at[pl.ds(j * 128, 128)], sem)
        for j in range(NCH):
            pltpu.make_async_copy(
                wt_hbm.at[idx_h.at[0]],
                rows_t.at[pl.ds(0, 128)], sem).wait()

        def dot_group(j, g, _):
            res = jnp.zeros((16,), jnp.float32)
            ht_vec = (idx_t[j, pl.ds(g * 16, 16)] & 1) * 64
            hc_vec = (idx_c[j, pl.ds(g * 16, 16)] & 1) * 64
            for i in range(16):
                p = g * 16 + i
                b = j * 128 + p
                ht = ht_vec[i]
                hc = hc_vec[i]
                acc = None
                for dc in range(D // 16):
                    we = rows_t[b, pl.ds(ht + dc * 16, 16)]
                    ce = rows_c[p, pl.ds(hc + dc * 16, 16)]
                    acc = we * ce if acc is None else acc + we * ce
                res = jnp.where(lanes == i, jnp.sum(acc), res)
            dots_v[j, pl.ds(g * 16, 16)] = res
            return _

        def chunk_body(j, _):
            pltpu.async_copy(wc_hbm.at[idx_h.at[j]], rows_c, sem)
            pltpu.make_async_copy(
                wc_hbm.at[idx_h.at[0]], rows_c, sem).wait()
            lax.fori_loop(0, 128 // 16,
                          lambda g, c: dot_group(j, g, c), 0)
            return _

        for n in range(NN):
            pltpu.sync_copy(ctx_hbm.at[n, pl.ds(crow, NCH)], idx_c)
            halve(idx_c, idx_h)
            lax.fori_loop(0, NCH, chunk_body, 0)
            pltpu.sync_copy(dots_v, out_hbm.at[n, pl.ds(crow, NCH)])

    return k


_k = _sc_kernel()


def kernel(target, context, W_target, W_context):
    tgt2 = target.reshape(B // 128, 128)
    ctx3 = context.reshape(B, NN).T.reshape(NN, B // 128, 128)
    wt2 = W_target.reshape(V // 2, 2 * D)
    wc2 = W_context.reshape(V // 2, 2 * D)
    out = _k(tgt2, ctx3, wt2, wc2)
    return out.reshape(NN, B).T


# speedup vs baseline: 1.5485x; 1.5485x over previous
import functools

import jax
import jax.numpy as jnp
from jax import lax
from jax.experimental import pallas as pl
from jax.experimental.pallas import tpu as pltpu
from jax.experimental.pallas import tpu_sc as plsc

V = 1000000
D = 64
B = 16384
NN = 5
NW = 32
BPW = B // NW
NCH = BPW // 128


def _sc_kernel():
    mesh = plsc.VectorSubcoreMesh(core_axis_name="c", subcore_axis_name="s")

    @functools.partial(
        pl.kernel,
        mesh=mesh,
        compiler_params=pltpu.CompilerParams(needs_layout_passes=False),
        out_type=jax.ShapeDtypeStruct((NN, B // 128, 128), jnp.float32),
        scratch_types=[
            pltpu.VMEM((NCH, 128), jnp.int32),
            pltpu.VMEM((NCH, 128), jnp.int32),
            pltpu.VMEM((BPW, D), jnp.float32),
            pltpu.VMEM((128, D), jnp.float32),
            pltpu.VMEM((NCH, 128), jnp.float32),
            pltpu.SemaphoreType.DMA,
        ],
    )
    def k(tgt_hbm, ctx_hbm, wt_hbm, wc_hbm, out_hbm, idx_t, idx_c,
          rows_t, rows_c, dots_v, sem):
        wid = lax.axis_index("s") * 2 + lax.axis_index("c")
        crow = wid * NCH
        lanes = lax.iota(jnp.int32, 16)

        def fetch_group(table, idx_ref, rows_ref, row_of_g):
            def body(g, _):
                v = idx_ref[g >> 3, pl.ds((g & 7) * 16, 16)]
                base = row_of_g(g)
                for i in range(16):
                    pltpu.async_copy(
                        table.at[pl.ds(v[i], 1)],
                        rows_ref.at[pl.ds(base + i, 1)], sem)
                return _
            return body

        pltpu.sync_copy(tgt_hbm.at[pl.ds(crow, NCH)], idx_t)
        lax.fori_loop(0, BPW // 16,
                      fetch_group(wt_hbm, idx_t, rows_t, lambda g: g * 16), 0)
        pltpu.make_async_copy(wt_hbm.at[pl.ds(0, BPW)], rows_t, sem).wait()

        def dot_group(j, g, _):
            res = jnp.zeros((16,), jnp.float32)
            for i in range(16):
                p = g * 16 + i
                b = j * 128 + p
                acc = None
                for dc in range(D // 16):
                    we = rows_t[b, pl.ds(dc * 16, 16)]
                    ce = rows_c[p, pl.ds(dc * 16, 16)]
                    acc = we * ce if acc is None else acc + we * ce
                res = jnp.where(lanes == i, jnp.sum(acc), res)
            dots_v[j, pl.ds(g * 16, 16)] = res
            return _

        def chunk_body(j, _):
            def issue(g, c):
                return fetch_group(wc_hbm, idx_c, rows_c,
                                   lambda gg: (gg & 7) * 16)(g, c)
            lax.fori_loop(j * 8, j * 8 + 8, issue, 0)
            pltpu.make_async_copy(
                wc_hbm.at[pl.ds(0, 128)], rows_c, sem).wait()
            lax.fori_loop(0, 128 // 16,
                          lambda g, c: dot_group(j, g, c), 0)
            return _

        for n in range(NN):
            pltpu.sync_copy(ctx_hbm.at[n, pl.ds(crow, NCH)], idx_c)
            lax.fori_loop(0, NCH, chunk_body, 0)
            pltpu.sync_copy(dots_v, out_hbm.at[n, pl.ds(crow, NCH)])

    return k


_k = _sc_kernel()


def kernel(target, context, W_target, W_context):
    tgt2 = target.reshape(B // 128, 128)
    ctx3 = context.reshape(B, NN).T.reshape(NN, B // 128, 128)
    out = _k(tgt2, ctx3, W_target, W_context)
    return out.reshape(NN, B).T
